# baseline (speedup 1.0000x reference)
import functools

import jax
import jax.numpy as jnp
from jax import lax
from jax.experimental import pallas as pl
from jax.experimental.pallas import tpu as pltpu
from jax.experimental.pallas import tpu_sc as plsc

_N_NODES = 10000
_N_REL2 = 512
_N_EDGES = 320000
_H = 128

_NC = 2
_NS = 16
_E_SC = _N_EDGES // _NC
_E_TILE = _E_SC // _NS
_CHUNKS = 5
_CH_NODES = 2048
_N_PAD = _CHUNKS * _CH_NODES
_CH_WORDS = _CH_NODES * _N_REL2
_SLICE = _CH_WORDS // _NS
_GARB = 4096
_PAD_E = 10112
_ZB = 8192
_NBLK = _N_PAD // 128


def _hist_body(dst_hbm, relid_hbm, c_hbm, buf_d, buf_r, idx1, ones1, zb, shared):
    c = lax.axis_index("c")
    s = lax.axis_index("s")
    base = c * _E_SC + s * _E_TILE
    pltpu.sync_copy(dst_hbm.at[pl.ds(base, _E_TILE)], buf_d)
    pltpu.sync_copy(relid_hbm.at[pl.ds(base, _E_TILE)], buf_r)

    one16 = jnp.full((16,), 1.0, jnp.float32)

    def f_ones(i, carry):
        ones1[pl.ds(i * 16, 16)] = one16
        return carry

    lax.fori_loop(0, _PAD_E // 16, f_ones, 0)

    def f_flat(i, carry):
        d = buf_d[pl.ds(i * 16, 16)]
        r = buf_r[pl.ds(i * 16, 16)]
        buf_d[pl.ds(i * 16, 16)] = d * _N_REL2 + r
        return carry

    lax.fori_loop(0, _E_TILE // 16, f_flat, 0)

    zero16 = jnp.zeros((16,), jnp.float32)

    def f_zb(i, carry):
        zb[pl.ds(i * 16, 16)] = zero16
        return carry

    lax.fori_loop(0, _ZB // 16, f_zb, 0)

    lanes = lax.iota(jnp.int32, 16)

    def f_pad(i, carry):
        j = _E_TILE + i * 16
        idx1[pl.ds(j, 16)] = _CH_WORDS + ((j + lanes) & (_GARB - 1))
        return carry

    lax.fori_loop(0, (_PAD_E - _E_TILE) // 16, f_pad, 0)

    for ch in range(_CHUNKS):
        flo = ch * _CH_WORDS
        for z in range(_SLICE // _ZB):
            pltpu.sync_copy(zb, shared.at[pl.ds(s * _SLICE + z * _ZB, _ZB)])
        plsc.subcore_barrier()

        def f_idx(i, carry):
            f = buf_d[pl.ds(i * 16, 16)] - flo
            m = (f >= 0) & (f < _CH_WORDS)
            dloc = lax.shift_right_logical(f, 9)
            r = f & (_N_REL2 - 1)
            tiled = (
                lax.shift_left(lax.shift_right_logical(dloc, 3), 12)
                + lax.shift_left(lax.shift_right_logical(r, 7), 10)
                + lax.shift_left(dloc & 7, 7)
                + (r & 127)
            )
            gi = _CH_WORDS + (f & (_GARB - 1))
            idx1[pl.ds(i * 16, 16)] = jnp.where(m, tiled, gi)
            return carry

        lax.fori_loop(0, _E_TILE // 16, f_idx, 0)

        pltpu.sync_copy(ones1, shared.at[idx1], add=True)
        plsc.subcore_barrier()
        out_base = (c * _NBLK + ch * _NS + s) * _SLICE
        pltpu.sync_copy(shared.at[pl.ds(s * _SLICE, _SLICE)],
                        c_hbm.at[pl.ds(out_base, _SLICE)])


_hist = pl.kernel(
    _hist_body,
    out_type=jax.ShapeDtypeStruct((_NC * _NBLK * _SLICE,), jnp.float32),
    mesh=plsc.VectorSubcoreMesh(core_axis_name="c", subcore_axis_name="s"),
    scratch_types=[
        pltpu.VMEM((_E_TILE,), jnp.int32),
        pltpu.VMEM((_E_TILE,), jnp.int32),
        pltpu.VMEM((_PAD_E,), jnp.int32),
        pltpu.VMEM((_PAD_E,), jnp.float32),
        pltpu.VMEM((_ZB,), jnp.float32),
        pltpu.VMEM_SHARED((_CH_WORDS + _GARB,), jnp.float32),
    ],
)


def _dense_body(ent_ref, rel_ref, w_ref, c0_ref, c1_ref, out_ref):
    hi = lax.Precision.HIGHEST
    ent = ent_ref[...]
    rel = rel_ref[...]
    ent_h = ent.astype(jnp.bfloat16)
    rel_h = rel.astype(jnp.bfloat16)
    ent_l = (ent - ent_h.astype(jnp.float32)).astype(jnp.bfloat16)
    rel_l = (rel - rel_h.astype(jnp.float32)).astype(jnp.bfloat16)
    dn = (((1,), (1,)), ((), ()))

    def bmm(x, y):
        return lax.dot_general(x, y, dn, precision=lax.Precision.DEFAULT,
                               preferred_element_type=jnp.float32)

    s = bmm(ent_h, rel_h) + (bmm(ent_h, rel_l) + bmm(ent_l, rel_h))
    cb = (c0_ref[0] + c1_ref[0]).reshape(_BN, _N_REL2)
    m = jnp.max(jnp.where(cb > 0, s, -jnp.inf), axis=1, keepdims=True)
    m = jnp.where(jnp.isfinite(m), m, 0.0)
    a = cb * jnp.exp(jnp.minimum(s - m, 0.0))
    denom = jnp.sum(a, axis=1, keepdims=True)
    h = lax.dot_general(a, rel, (((1,), (0,)), ((), ())),
                        precision=lax.Precision.DEFAULT, preferred_element_type=jnp.float32)
    neigh = h / (denom + 1e-16)
    out_ref[...] = jnp.tanh(
        lax.dot_general(neigh, w_ref[...], (((1,), (0,)), ((), ())),
                        precision=lax.Precision.DEFAULT,
                        preferred_element_type=jnp.float32))


_BN = 1024


def _dense(ent_emb, rel_emb, neigh_w, counts):
    return pl.pallas_call(
        _dense_body,
        grid=(_N_PAD // _BN,),
        in_specs=[
            pl.BlockSpec((_BN, _H), lambda i: (i, 0)),
            pl.BlockSpec((_N_REL2, _H), lambda i: (0, 0)),
            pl.BlockSpec((_H, _H), lambda i: (0, 0)),
            pl.BlockSpec((1, _BN // 128, 128, _N_REL2), lambda i: (0, i, 0, 0)),
            pl.BlockSpec((1, _BN // 128, 128, _N_REL2), lambda i: (1, i, 0, 0)),
        ],
        out_specs=pl.BlockSpec((_BN, _H), lambda i: (i, 0)),
        out_shape=jax.ShapeDtypeStruct((_N_PAD, _H), jnp.float32),
    )(ent_emb, rel_emb, neigh_w, counts, counts)


@jax.jit
def kernel(ent_emb, rel_emb, neigh_w, edge_index, rel_id):
    cp = _hist(edge_index[1], rel_id).reshape(_NC, _NBLK, 128, _N_REL2)
    ent_pad = jnp.concatenate(
        [ent_emb, jnp.zeros((_N_PAD - _N_NODES, _H), jnp.float32)])
    out = _dense(ent_pad, rel_emb, neigh_w, cp)
    return out[:_N_NODES]


# speedup vs baseline: 1.3151x; 1.3151x over previous
import functools

import jax
import jax.numpy as jnp
from jax import lax
from jax.experimental import pallas as pl
from jax.experimental.pallas import tpu as pltpu
from jax.experimental.pallas import tpu_sc as plsc

_N_NODES = 10000
_N_REL2 = 512
_N_EDGES = 320000
_H = 128

_NC = 2
_NS = 16
_E_SC = _N_EDGES // _NC
_E_TILE = _E_SC // _NS
_CHUNKS = 5
_CH_NODES = 2048
_N_PAD = _CHUNKS * _CH_NODES
_CH_WORDS = _CH_NODES * _N_REL2
_SLICE = _CH_WORDS // _NS
_GARB = 4096
_PAD_E = 10112
_ZB = 8192
_NBLK = _N_PAD // 128
_QSL = 128 * 128


def _hist_body(dst_hbm, relid_hbm, c_hbm, buf_d, buf_r, idx1, ones1, zb, shared):
    c = lax.axis_index("c")
    s = lax.axis_index("s")
    base = c * _E_SC + s * _E_TILE
    pltpu.sync_copy(dst_hbm.at[pl.ds(base, _E_TILE)], buf_d)
    pltpu.sync_copy(relid_hbm.at[pl.ds(base, _E_TILE)], buf_r)

    one16 = jnp.full((16,), 1.0, jnp.float32)

    def f_ones(i, carry):
        ones1[pl.ds(i * 16, 16)] = one16
        return carry

    lax.fori_loop(0, _PAD_E // 16, f_ones, 0)

    def f_flat(i, carry):
        d = buf_d[pl.ds(i * 16, 16)]
        r = buf_r[pl.ds(i * 16, 16)]
        buf_d[pl.ds(i * 16, 16)] = d * _N_REL2 + r
        return carry

    lax.fori_loop(0, _E_TILE // 16, f_flat, 0)

    zero16 = jnp.zeros((16,), jnp.float32)

    def f_zb(i, carry):
        zb[pl.ds(i * 16, 16)] = zero16
        return carry

    lax.fori_loop(0, _ZB // 16, f_zb, 0)

    lanes = lax.iota(jnp.int32, 16)

    def f_pad(i, carry):
        j = _E_TILE + i * 16
        idx1[pl.ds(j, 16)] = _CH_WORDS + ((j + lanes) & (_GARB - 1))
        return carry

    lax.fori_loop(0, (_PAD_E - _E_TILE) // 16, f_pad, 0)

    for ch in range(_CHUNKS):
        flo = ch * _CH_WORDS
        for z in range(_SLICE // _ZB):
            pltpu.sync_copy(zb, shared.at[pl.ds(s * _SLICE + z * _ZB, _ZB)])
        plsc.subcore_barrier()

        def f_idx(i, carry):
            f = buf_d[pl.ds(i * 16, 16)] - flo
            m = (f >= 0) & (f < _CH_WORDS)
            qoff = (
                lax.shift_left((lax.shift_right_logical(f, 7)) & 3, 18)
                + lax.shift_left(lax.shift_right_logical(f, 9), 7)
                + (f & 127)
            )
            gi = _CH_WORDS + (f & (_GARB - 1))
            idx1[pl.ds(i * 16, 16)] = jnp.where(m, qoff, gi)
            return carry

        lax.fori_loop(0, _E_TILE // 16, f_idx, 0)

        pltpu.sync_copy(ones1, shared.at[idx1], add=True)
        plsc.subcore_barrier()
        for q in range(4):
            src_off = q * (_CH_NODES * 128) + s * _QSL
            dst_off = ((c * 4 + q) * _N_PAD + ch * _CH_NODES + s * 128) * 128
            pltpu.sync_copy(shared.at[pl.ds(src_off, _QSL)],
                            c_hbm.at[pl.ds(dst_off, _QSL)])


_hist = pl.kernel(
    _hist_body,
    out_type=jax.ShapeDtypeStruct((_NC * _NBLK * _SLICE,), jnp.float32),
    mesh=plsc.VectorSubcoreMesh(core_axis_name="c", subcore_axis_name="s"),
    scratch_types=[
        pltpu.VMEM((_E_TILE,), jnp.int32),
        pltpu.VMEM((_E_TILE,), jnp.int32),
        pltpu.VMEM((_PAD_E,), jnp.int32),
        pltpu.VMEM((_PAD_E,), jnp.float32),
        pltpu.VMEM((_ZB,), jnp.float32),
        pltpu.VMEM_SHARED((_CH_WORDS + _GARB,), jnp.float32),
    ],
)


def _dense_body(ent_ref, rel_ref, w_ref, *c_refs_and_out):
    c_refs = c_refs_and_out[:-1]
    out_ref = c_refs_and_out[-1]
    dflt = lax.Precision.DEFAULT
    ent = ent_ref[...]
    ent_h = ent.astype(jnp.bfloat16)
    ent_l = (ent - ent_h.astype(jnp.float32)).astype(jnp.bfloat16)
    dn = (((1,), (1,)), ((), ()))

    cq, sq = [], []
    neg_inf = jnp.float32(-jnp.inf)
    m = None
    for q in range(4):
        rel_q = rel_ref[pl.ds(q * _H, _H), :]
        rq_h = rel_q.astype(jnp.bfloat16)
        rq_l = (rel_q - rq_h.astype(jnp.float32)).astype(jnp.bfloat16)
        s = lax.dot_general(ent_h, rq_h, dn, precision=dflt,
                            preferred_element_type=jnp.float32)
        s = s + (lax.dot_general(ent_h, rq_l, dn, precision=dflt,
                                 preferred_element_type=jnp.float32)
                 + lax.dot_general(ent_l, rq_h, dn, precision=dflt,
                                   preferred_element_type=jnp.float32))
        c = c_refs[q][0, 0] + c_refs[4 + q][0, 0]
        mq = jnp.max(jnp.where(c > 0, s, neg_inf), axis=1, keepdims=True)
        m = mq if m is None else jnp.maximum(m, mq)
        cq.append(c)
        sq.append(s)

    m = jnp.where(jnp.isfinite(m), m, 0.0)
    h = None
    denom = None
    for q in range(4):
        rel_q = rel_ref[pl.ds(q * _H, _H), :]
        a = cq[q] * jnp.exp(jnp.minimum(sq[q] - m, 0.0))
        dq = jnp.sum(a, axis=1, keepdims=True)
        denom = dq if denom is None else denom + dq
        hq = lax.dot_general(a, rel_q, (((1,), (0,)), ((), ())),
                             precision=dflt, preferred_element_type=jnp.float32)
        h = hq if h is None else h + hq
    neigh = h / (denom + 1e-16)
    out_ref[...] = jnp.tanh(
        lax.dot_general(neigh, w_ref[...], (((1,), (0,)), ((), ())),
                        precision=dflt, preferred_element_type=jnp.float32))


_BN = 1024


def _dense(ent_emb, rel_emb, neigh_w, counts):
    return pl.pallas_call(
        _dense_body,
        grid=(_N_PAD // _BN,),
        in_specs=[
            pl.BlockSpec((_BN, _H), lambda i: (i, 0)),
            pl.BlockSpec((_N_REL2, _H), lambda i: (0, 0)),
            pl.BlockSpec((_H, _H), lambda i: (0, 0)),
        ] + [
            pl.BlockSpec((1, 1, _BN, 128),
                         (lambda c, q: (lambda i: (c, q, i, 0)))(c, q))
            for c in range(2) for q in range(4)
        ],
        out_specs=pl.BlockSpec((_BN, _H), lambda i: (i, 0)),
        out_shape=jax.ShapeDtypeStruct((_N_PAD, _H), jnp.float32),
    )(ent_emb, rel_emb, neigh_w, *([counts] * 8))


@jax.jit
def kernel(ent_emb, rel_emb, neigh_w, edge_index, rel_id):
    cp = _hist(edge_index[1], rel_id).reshape(_NC, 4, _N_PAD, 128)
    ent_pad = jnp.concatenate(
        [ent_emb, jnp.zeros((_N_PAD - _N_NODES, _H), jnp.float32)])
    out = _dense(ent_pad, rel_emb, neigh_w, cp)
    return out[:_N_NODES]


# speedup vs baseline: 1.3233x; 1.0062x over previous
import functools

import jax
import jax.numpy as jnp
from jax import lax
from jax.experimental import pallas as pl
from jax.experimental.pallas import tpu as pltpu
from jax.experimental.pallas import tpu_sc as plsc

_N_NODES = 10000
_N_REL2 = 512
_N_EDGES = 320000
_H = 128

_NC = 2
_NS = 16
_E_SC = _N_EDGES // _NC
_E_TILE = _E_SC // _NS
_CHUNKS = 5
_CH_NODES = 2048
_N_PAD = _CHUNKS * _CH_NODES
_CH_WORDS = _CH_NODES * _N_REL2
_SLICE = _CH_WORDS // _NS
_GARB = 4096
_PAD_E = 10112
_ZB = 8192
_NBLK = _N_PAD // 128
_QSL = 128 * 128


def _hist_body(dst_hbm, relid_hbm, c_hbm, buf_d, buf_r, idx1, ones1, zb, shared):
    c = lax.axis_index("c")
    s = lax.axis_index("s")
    base = c * _E_SC + s * _E_TILE
    pltpu.sync_copy(dst_hbm.at[pl.ds(base, _E_TILE)], buf_d)
    pltpu.sync_copy(relid_hbm.at[pl.ds(base, _E_TILE)], buf_r)

    one16 = jnp.full((16,), 1.0, jnp.float32)

    def f_ones(i, carry):
        ones1[pl.ds(i * 16, 16)] = one16
        return carry

    lax.fori_loop(0, _PAD_E // 16, f_ones, 0)

    def f_flat(i, carry):
        d = buf_d[pl.ds(i * 16, 16)]
        r = buf_r[pl.ds(i * 16, 16)]
        buf_d[pl.ds(i * 16, 16)] = d * _N_REL2 + r
        return carry

    lax.fori_loop(0, _E_TILE // 16, f_flat, 0)

    zero16 = jnp.zeros((16,), jnp.float32)

    def f_zb(i, carry):
        zb[pl.ds(i * 16, 16)] = zero16
        return carry

    lax.fori_loop(0, _ZB // 16, f_zb, 0)

    lanes = lax.iota(jnp.int32, 16)

    def f_pad(i, carry):
        j = _E_TILE + i * 16
        idx1[pl.ds(j, 16)] = _CH_WORDS + ((j + lanes) & (_GARB - 1))
        return carry

    lax.fori_loop(0, (_PAD_E - _E_TILE) // 16, f_pad, 0)

    for ch in range(_CHUNKS):
        flo = ch * _CH_WORDS
        for z in range(_SLICE // _ZB):
            pltpu.sync_copy(zb, shared.at[pl.ds(s * _SLICE + z * _ZB, _ZB)])
        plsc.subcore_barrier()

        def f_idx(i, carry):
            f = buf_d[pl.ds(i * 16, 16)] - flo
            m = (f >= 0) & (f < _CH_WORDS)
            qoff = (
                lax.shift_left((lax.shift_right_logical(f, 7)) & 3, 18)
                + lax.shift_left(lax.shift_right_logical(f, 9), 7)
                + (f & 127)
            )
            gi = _CH_WORDS + (f & (_GARB - 1))
            idx1[pl.ds(i * 16, 16)] = jnp.where(m, qoff, gi)
            return carry

        lax.fori_loop(0, _E_TILE // 16, f_idx, 0)

        pltpu.sync_copy(ones1, shared.at[idx1], add=True)
        plsc.subcore_barrier()
        for q in range(4):
            src_off = q * (_CH_NODES * 128) + s * _QSL
            dst_off = ((c * 4 + q) * _N_PAD + ch * _CH_NODES + s * 128) * 128
            pltpu.sync_copy(shared.at[pl.ds(src_off, _QSL)],
                            c_hbm.at[pl.ds(dst_off, _QSL)])


_hist = pl.kernel(
    _hist_body,
    out_type=jax.ShapeDtypeStruct((_NC * _NBLK * _SLICE,), jnp.float32),
    mesh=plsc.VectorSubcoreMesh(core_axis_name="c", subcore_axis_name="s"),
    scratch_types=[
        pltpu.VMEM((_E_TILE,), jnp.int32),
        pltpu.VMEM((_E_TILE,), jnp.int32),
        pltpu.VMEM((_PAD_E,), jnp.int32),
        pltpu.VMEM((_PAD_E,), jnp.float32),
        pltpu.VMEM((_ZB,), jnp.float32),
        pltpu.VMEM_SHARED((_CH_WORDS + _GARB,), jnp.float32),
    ],
)


def _smat_body(ent_ref, rel_ref, s_ref):
    dflt = lax.Precision.DEFAULT
    ent = ent_ref[...]
    ent_h = ent.astype(jnp.bfloat16)
    ent_l = (ent - ent_h.astype(jnp.float32)).astype(jnp.bfloat16)
    rel = rel_ref[...]
    rel_h = rel.astype(jnp.bfloat16)
    rel_l = (rel - rel_h.astype(jnp.float32)).astype(jnp.bfloat16)
    dn = (((1,), (1,)), ((), ()))
    s = lax.dot_general(ent_h, rel_h, dn, precision=dflt,
                        preferred_element_type=jnp.float32)
    s_ref[...] = s + (
        lax.dot_general(ent_h, rel_l, dn, precision=dflt,
                        preferred_element_type=jnp.float32)
        + lax.dot_general(ent_l, rel_h, dn, precision=dflt,
                          preferred_element_type=jnp.float32))


def _smat(ent_pad, rel_emb):
    return pl.pallas_call(
        _smat_body,
        grid=(_N_PAD // _BN,),
        in_specs=[
            pl.BlockSpec((_BN, _H), lambda i: (i, 0)),
            pl.BlockSpec((_N_REL2, _H), lambda i: (0, 0)),
        ],
        out_specs=pl.BlockSpec((_BN, _N_REL2), lambda i: (i, 0)),
        out_shape=jax.ShapeDtypeStruct((_N_PAD, _N_REL2), jnp.float32),
    )(ent_pad, rel_emb)


def _dense_body(s_ref, rel_ref, w_ref, *c_refs_and_out):
    c_refs = c_refs_and_out[:-1]
    out_ref = c_refs_and_out[-1]
    dflt = lax.Precision.DEFAULT

    cq, sq = [], []
    neg_inf = jnp.float32(-jnp.inf)
    m = None
    for q in range(4):
        s = s_ref[:, pl.ds(q * _H, _H)]
        c = c_refs[q][0, 0] + c_refs[4 + q][0, 0]
        mq = jnp.max(jnp.where(c > 0, s, neg_inf), axis=1, keepdims=True)
        m = mq if m is None else jnp.maximum(m, mq)
        cq.append(c)
        sq.append(s)

    m = jnp.where(jnp.isfinite(m), m, 0.0)
    h = None
    denom = None
    for q in range(4):
        rel_q = rel_ref[pl.ds(q * _H, _H), :]
        a = cq[q] * jnp.exp(jnp.minimum(sq[q] - m, 0.0))
        dq = jnp.sum(a, axis=1, keepdims=True)
        denom = dq if denom is None else denom + dq
        hq = lax.dot_general(a, rel_q, (((1,), (0,)), ((), ())),
                             precision=dflt, preferred_element_type=jnp.float32)
        h = hq if h is None else h + hq
    neigh = h / (denom + 1e-16)
    out_ref[...] = jnp.tanh(
        lax.dot_general(neigh, w_ref[...], (((1,), (0,)), ((), ())),
                        precision=dflt, preferred_element_type=jnp.float32))


_BN = 1024


def _dense(s_all, rel_emb, neigh_w, counts):
    return pl.pallas_call(
        _dense_body,
        grid=(_N_PAD // _BN,),
        in_specs=[
            pl.BlockSpec((_BN, _N_REL2), lambda i: (i, 0)),
            pl.BlockSpec((_N_REL2, _H), lambda i: (0, 0)),
            pl.BlockSpec((_H, _H), lambda i: (0, 0)),
        ] + [
            pl.BlockSpec((1, 1, _BN, 128),
                         (lambda c, q: (lambda i: (c, q, i, 0)))(c, q))
            for c in range(2) for q in range(4)
        ],
        out_specs=pl.BlockSpec((_BN, _H), lambda i: (i, 0)),
        out_shape=jax.ShapeDtypeStruct((_N_PAD, _H), jnp.float32),
    )(s_all, rel_emb, neigh_w, *([counts] * 8))


@jax.jit
def kernel(ent_emb, rel_emb, neigh_w, edge_index, rel_id):
    cp = _hist(edge_index[1], rel_id).reshape(_NC, 4, _N_PAD, 128)
    ent_pad = jnp.concatenate(
        [ent_emb, jnp.zeros((_N_PAD - _N_NODES, _H), jnp.float32)])
    s_all = _smat(ent_pad, rel_emb)
    out = _dense(s_all, rel_emb, neigh_w, cp)
    return out[:_N_NODES]


# speedup vs baseline: 1.3768x; 1.0405x over previous
import functools

import jax
import jax.numpy as jnp
from jax import lax
from jax.experimental import pallas as pl
from jax.experimental.pallas import tpu as pltpu
from jax.experimental.pallas import tpu_sc as plsc

_N_NODES = 10000
_N_REL2 = 512
_N_EDGES = 320000
_H = 128

_NC = 2
_NS = 16
_E_SC = _N_EDGES // _NC
_E_TILE = _E_SC // _NS
_CHUNKS = 5
_CH_NODES = 2048
_N_PAD = _CHUNKS * _CH_NODES
_CH_WORDS = _CH_NODES * _N_REL2
_SLICE = _CH_WORDS // _NS
_GARB = 4096
_PAD_E = 10112
_ZB = 8192
_NBLK = _N_PAD // 128
_QSL = 128 * 128
_E_WIN = _E_TILE + 240


def _hist_body(edge_hbm, relid_hbm, c_hbm, buf2, buf_r, idx1, ones1, zb, shared):
    c = lax.axis_index("c")
    s = lax.axis_index("s")
    base = c * _E_SC + s * _E_TILE
    off = s * 16
    pltpu.sync_copy(
        edge_hbm.at[:, pl.ds(pl.multiple_of(base - off, 128), _E_WIN)], buf2)
    pltpu.sync_copy(relid_hbm.at[pl.ds(base, _E_TILE)], buf_r)

    one16 = jnp.full((16,), 1.0, jnp.float32)

    def f_ones(i, carry):
        ones1[pl.ds(i * 16, 16)] = one16
        return carry

    lax.fori_loop(0, _PAD_E // 16, f_ones, 0)

    def f_flat(i, carry):
        d = buf2[1, pl.ds(off + i * 16, 16)]
        r = buf_r[pl.ds(i * 16, 16)]
        buf2[1, pl.ds(off + i * 16, 16)] = d * _N_REL2 + r
        return carry

    lax.fori_loop(0, _E_TILE // 16, f_flat, 0)

    zero16 = jnp.zeros((16,), jnp.float32)

    def f_zb(i, carry):
        zb[pl.ds(i * 16, 16)] = zero16
        return carry

    lax.fori_loop(0, _ZB // 16, f_zb, 0)

    lanes = lax.iota(jnp.int32, 16)

    def f_pad(i, carry):
        j = _E_TILE + i * 16
        idx1[pl.ds(j, 16)] = _CH_WORDS + ((j + lanes) & (_GARB - 1))
        return carry

    lax.fori_loop(0, (_PAD_E - _E_TILE) // 16, f_pad, 0)

    for ch in range(_CHUNKS):
        flo = ch * _CH_WORDS
        for z in range(_SLICE // _ZB):
            pltpu.sync_copy(zb, shared.at[pl.ds(s * _SLICE + z * _ZB, _ZB)])
        plsc.subcore_barrier()

        def f_idx(i, carry):
            f = buf2[1, pl.ds(off + i * 16, 16)] - flo
            m = (f >= 0) & (f < _CH_WORDS)
            qoff = (
                lax.shift_left((lax.shift_right_logical(f, 7)) & 3, 18)
                + lax.shift_left(lax.shift_right_logical(f, 9), 7)
                + (f & 127)
            )
            gi = _CH_WORDS + (f & (_GARB - 1))
            idx1[pl.ds(i * 16, 16)] = jnp.where(m, qoff, gi)
            return carry

        lax.fori_loop(0, _E_TILE // 16, f_idx, 0)

        pltpu.sync_copy(ones1, shared.at[idx1], add=True)
        plsc.subcore_barrier()
        for q in range(4):
            src_off = q * (_CH_NODES * 128) + s * _QSL
            dst_off = ((c * 4 + q) * _N_PAD + ch * _CH_NODES + s * 128) * 128
            pltpu.sync_copy(shared.at[pl.ds(src_off, _QSL)],
                            c_hbm.at[pl.ds(dst_off, _QSL)])


_hist = pl.kernel(
    _hist_body,
    out_type=jax.ShapeDtypeStruct((_NC * _NBLK * _SLICE,), jnp.float32),
    mesh=plsc.VectorSubcoreMesh(core_axis_name="c", subcore_axis_name="s"),
    scratch_types=[
        pltpu.VMEM((2, _E_WIN), jnp.int32),
        pltpu.VMEM((_E_TILE,), jnp.int32),
        pltpu.VMEM((_PAD_E,), jnp.int32),
        pltpu.VMEM((_PAD_E,), jnp.float32),
        pltpu.VMEM((_ZB,), jnp.float32),
        pltpu.VMEM_SHARED((_CH_WORDS + _GARB,), jnp.float32),
    ],
)


def _smat_body(ent_ref, rel_ref, s_ref):
    dflt = lax.Precision.DEFAULT
    ent = ent_ref[...]
    ent_h = ent.astype(jnp.bfloat16)
    ent_l = (ent - ent_h.astype(jnp.float32)).astype(jnp.bfloat16)
    rel = rel_ref[...]
    rel_h = rel.astype(jnp.bfloat16)
    rel_l = (rel - rel_h.astype(jnp.float32)).astype(jnp.bfloat16)
    dn = (((1,), (1,)), ((), ()))
    s = lax.dot_general(ent_h, rel_h, dn, precision=dflt,
                        preferred_element_type=jnp.float32)
    s_ref[...] = s + (
        lax.dot_general(ent_h, rel_l, dn, precision=dflt,
                        preferred_element_type=jnp.float32)
        + lax.dot_general(ent_l, rel_h, dn, precision=dflt,
                          preferred_element_type=jnp.float32))


def _smat(ent_pad, rel_emb):
    return pl.pallas_call(
        _smat_body,
        grid=(_N_PAD // _BN,),
        in_specs=[
            pl.BlockSpec((_BN, _H), lambda i: (i, 0)),
            pl.BlockSpec((_N_REL2, _H), lambda i: (0, 0)),
        ],
        out_specs=pl.BlockSpec((_BN, _N_REL2), lambda i: (i, 0)),
        out_shape=jax.ShapeDtypeStruct((_N_PAD, _N_REL2), jnp.float32),
    )(ent_pad, rel_emb)


def _dense_body(s_ref, rel_ref, w_ref, *c_refs_and_out):
    c_refs = c_refs_and_out[:-1]
    out_ref = c_refs_and_out[-1]
    dflt = lax.Precision.DEFAULT

    cq, sq = [], []
    neg_inf = jnp.float32(-jnp.inf)
    m = None
    for q in range(4):
        s = s_ref[:, pl.ds(q * _H, _H)]
        c = c_refs[q][0, 0] + c_refs[4 + q][0, 0]
        mq = jnp.max(jnp.where(c > 0, s, neg_inf), axis=1, keepdims=True)
        m = mq if m is None else jnp.maximum(m, mq)
        cq.append(c)
        sq.append(s)

    m = jnp.where(jnp.isfinite(m), m, 0.0)
    h = None
    denom = None
    for q in range(4):
        rel_q = rel_ref[pl.ds(q * _H, _H), :]
        a = cq[q] * jnp.exp(jnp.minimum(sq[q] - m, 0.0))
        dq = jnp.sum(a, axis=1, keepdims=True)
        denom = dq if denom is None else denom + dq
        hq = lax.dot_general(a, rel_q, (((1,), (0,)), ((), ())),
                             precision=dflt, preferred_element_type=jnp.float32)
        h = hq if h is None else h + hq
    neigh = h / (denom + 1e-16)
    out_ref[...] = jnp.tanh(
        lax.dot_general(neigh, w_ref[...], (((1,), (0,)), ((), ())),
                        precision=dflt, preferred_element_type=jnp.float32))


_BN = 1024


def _dense(s_all, rel_emb, neigh_w, counts):
    return pl.pallas_call(
        _dense_body,
        grid=(_N_PAD // _BN,),
        in_specs=[
            pl.BlockSpec((_BN, _N_REL2), lambda i: (i, 0)),
            pl.BlockSpec((_N_REL2, _H), lambda i: (0, 0)),
            pl.BlockSpec((_H, _H), lambda i: (0, 0)),
        ] + [
            pl.BlockSpec((1, 1, _BN, 128),
                         (lambda c, q: (lambda i: (c, q, i, 0)))(c, q))
            for c in range(2) for q in range(4)
        ],
        out_specs=pl.BlockSpec((_BN, _H), lambda i: (i, 0)),
        out_shape=jax.ShapeDtypeStruct((_N_NODES, _H), jnp.float32),
    )(s_all, rel_emb, neigh_w, *([counts] * 8))


@jax.jit
def kernel(ent_emb, rel_emb, neigh_w, edge_index, rel_id):
    cp = _hist(edge_index, rel_id).reshape(_NC, 4, _N_PAD, 128)
    s_all = _smat(ent_emb, rel_emb)
    return _dense(s_all, rel_emb, neigh_w, cp)


# speedup vs baseline: 1.5267x; 1.1089x over previous
import functools

import jax
import jax.numpy as jnp
from jax import lax
from jax.experimental import pallas as pl
from jax.experimental.pallas import tpu as pltpu
from jax.experimental.pallas import tpu_sc as plsc

_N_NODES = 10000
_N_REL2 = 512
_N_EDGES = 320000
_H = 128

_NC = 2
_NS = 16
_E_SC = _N_EDGES // _NC
_E_TILE = _E_SC // _NS
_CHUNKS = 5
_CH_NODES = 2048
_N_PAD = _CHUNKS * _CH_NODES
_CH_WORDS = _CH_NODES * _N_REL2
_SLICE = _CH_WORDS // _NS
_GARB = 4096
_PAD_E = 10112
_ZB = 8192
_NBLK = _N_PAD // 128
_QSL = 128 * 128
_E_WIN = _E_TILE + 240


def _hist_body(edge_hbm, relid_hbm, c_hbm, buf2, buf_r, idx1, ones1, zb, shared):
    c = lax.axis_index("c")
    s = lax.axis_index("s")
    base = c * _E_SC + s * _E_TILE
    off = s * 16
    pltpu.sync_copy(
        edge_hbm.at[:, pl.ds(pl.multiple_of(base - off, 128), _E_WIN)], buf2)
    pltpu.sync_copy(relid_hbm.at[pl.ds(base, _E_TILE)], buf_r)

    one16 = jnp.full((16,), 1.0, jnp.float32)

    def f_ones(i, carry):
        ones1[pl.ds(i * 16, 16)] = one16
        return carry

    lax.fori_loop(0, _PAD_E // 16, f_ones, 0)

    def f_flat(i, carry):
        d = buf2[1, pl.ds(off + i * 16, 16)]
        r = buf_r[pl.ds(i * 16, 16)]
        buf2[1, pl.ds(off + i * 16, 16)] = d * _N_REL2 + r
        return carry

    lax.fori_loop(0, _E_TILE // 16, f_flat, 0)

    zero16 = jnp.zeros((16,), jnp.float32)

    def f_zb(i, carry):
        zb[pl.ds(i * 16, 16)] = zero16
        return carry

    lax.fori_loop(0, _ZB // 16, f_zb, 0)

    lanes = lax.iota(jnp.int32, 16)

    def f_pad(i, carry):
        j = _E_TILE + i * 16
        idx1[pl.ds(j, 16)] = _CH_WORDS + ((j + lanes) & (_GARB - 1))
        return carry

    lax.fori_loop(0, (_PAD_E - _E_TILE) // 16, f_pad, 0)

    for ch in range(_CHUNKS):
        flo = ch * _CH_WORDS
        for z in range(_SLICE // _ZB):
            pltpu.sync_copy(zb, shared.at[pl.ds(s * _SLICE + z * _ZB, _ZB)])
        plsc.subcore_barrier()

        @plsc.parallel_loop(0, _E_TILE // 16, unroll=4)
        def f_idx(i):
            f = buf2[1, pl.ds(off + i * 16, 16)] - flo
            m = (f >= 0) & (f < _CH_WORDS)
            qoff = (
                lax.shift_left((lax.shift_right_logical(f, 7)) & 3, 18)
                + lax.shift_left(lax.shift_right_logical(f, 9), 7)
                + (f & 127)
            )
            gi = _CH_WORDS + (f & (_GARB - 1))
            idx1[pl.ds(i * 16, 16)] = jnp.where(m, qoff, gi)

        pltpu.sync_copy(ones1, shared.at[idx1], add=True)
        plsc.subcore_barrier()
        for q in range(4):
            src_off = q * (_CH_NODES * 128) + s * _QSL
            dst_off = ((c * 4 + q) * _N_PAD + ch * _CH_NODES + s * 128) * 128
            pltpu.sync_copy(shared.at[pl.ds(src_off, _QSL)],
                            c_hbm.at[pl.ds(dst_off, _QSL)])


_hist = pl.kernel(
    _hist_body,
    out_type=jax.ShapeDtypeStruct((_NC * _NBLK * _SLICE,), jnp.float32),
    mesh=plsc.VectorSubcoreMesh(core_axis_name="c", subcore_axis_name="s"),
    scratch_types=[
        pltpu.VMEM((2, _E_WIN), jnp.int32),
        pltpu.VMEM((_E_TILE,), jnp.int32),
        pltpu.VMEM((_PAD_E,), jnp.int32),
        pltpu.VMEM((_PAD_E,), jnp.float32),
        pltpu.VMEM((_ZB,), jnp.float32),
        pltpu.VMEM_SHARED((_CH_WORDS + _GARB,), jnp.float32),
    ],
)


def _smat_body(ent_ref, rel_ref, s_ref):
    dflt = lax.Precision.DEFAULT
    ent = ent_ref[...]
    ent_h = ent.astype(jnp.bfloat16)
    ent_l = (ent - ent_h.astype(jnp.float32)).astype(jnp.bfloat16)
    rel = rel_ref[...]
    rel_h = rel.astype(jnp.bfloat16)
    rel_l = (rel - rel_h.astype(jnp.float32)).astype(jnp.bfloat16)
    dn = (((1,), (1,)), ((), ()))
    s = lax.dot_general(ent_h, rel_h, dn, precision=dflt,
                        preferred_element_type=jnp.float32)
    s_ref[...] = s + (
        lax.dot_general(ent_h, rel_l, dn, precision=dflt,
                        preferred_element_type=jnp.float32)
        + lax.dot_general(ent_l, rel_h, dn, precision=dflt,
                          preferred_element_type=jnp.float32))


def _smat(ent_pad, rel_emb):
    return pl.pallas_call(
        _smat_body,
        grid=(_N_PAD // _BN,),
        in_specs=[
            pl.BlockSpec((_BN, _H), lambda i: (i, 0)),
            pl.BlockSpec((_N_REL2, _H), lambda i: (0, 0)),
        ],
        out_specs=pl.BlockSpec((_BN, _N_REL2), lambda i: (i, 0)),
        out_shape=jax.ShapeDtypeStruct((_N_PAD, _N_REL2), jnp.float32),
    )(ent_pad, rel_emb)


def _dense_body(s_ref, rel_ref, w_ref, *c_refs_and_out):
    c_refs = c_refs_and_out[:-1]
    out_ref = c_refs_and_out[-1]
    dflt = lax.Precision.DEFAULT

    cq, sq = [], []
    neg_inf = jnp.float32(-jnp.inf)
    m = None
    for q in range(4):
        s = s_ref[:, pl.ds(q * _H, _H)]
        c = c_refs[q][0, 0] + c_refs[4 + q][0, 0]
        mq = jnp.max(jnp.where(c > 0, s, neg_inf), axis=1, keepdims=True)
        m = mq if m is None else jnp.maximum(m, mq)
        cq.append(c)
        sq.append(s)

    m = jnp.where(jnp.isfinite(m), m, 0.0)
    h = None
    denom = None
    for q in range(4):
        rel_q = rel_ref[pl.ds(q * _H, _H), :]
        a = cq[q] * jnp.exp(jnp.minimum(sq[q] - m, 0.0))
        dq = jnp.sum(a, axis=1, keepdims=True)
        denom = dq if denom is None else denom + dq
        hq = lax.dot_general(a, rel_q, (((1,), (0,)), ((), ())),
                             precision=dflt, preferred_element_type=jnp.float32)
        h = hq if h is None else h + hq
    neigh = h / (denom + 1e-16)
    out_ref[...] = jnp.tanh(
        lax.dot_general(neigh, w_ref[...], (((1,), (0,)), ((), ())),
                        precision=dflt, preferred_element_type=jnp.float32))


_BN = 1024


def _dense(s_all, rel_emb, neigh_w, counts):
    return pl.pallas_call(
        _dense_body,
        grid=(_N_PAD // _BN,),
        in_specs=[
            pl.BlockSpec((_BN, _N_REL2), lambda i: (i, 0)),
            pl.BlockSpec((_N_REL2, _H), lambda i: (0, 0)),
            pl.BlockSpec((_H, _H), lambda i: (0, 0)),
        ] + [
            pl.BlockSpec((1, 1, _BN, 128),
                         (lambda c, q: (lambda i: (c, q, i, 0)))(c, q))
            for c in range(2) for q in range(4)
        ],
        out_specs=pl.BlockSpec((_BN, _H), lambda i: (i, 0)),
        out_shape=jax.ShapeDtypeStruct((_N_NODES, _H), jnp.float32),
    )(s_all, rel_emb, neigh_w, *([counts] * 8))


@jax.jit
def kernel(ent_emb, rel_emb, neigh_w, edge_index, rel_id):
    cp = _hist(edge_index, rel_id).reshape(_NC, 4, _N_PAD, 128)
    s_all = _smat(ent_emb, rel_emb)
    return _dense(s_all, rel_emb, neigh_w, cp)
